# baseline (speedup 1.0000x reference)
import jax
import jax.numpy as jnp
from jax import lax
from jax.experimental import pallas as pl
from jax.experimental.pallas import tpu as pltpu
from jax.experimental.pallas import tpu_sc as plsc

_IN_C = 192
_OUT_C = 384
_B = 4
_HW = 224 * 224
_W = 0.5

_NC = 2
_NS = 16
_NW = _NC * _NS
_CPW = (_B * _IN_C) // _NW
_GPB = _IN_C // _CPW

_NCHUNK = 4
_CL = _HW // _NCHUNK
_STEPS = _CPW * _NCHUNK
_DEPTH = 4
_NSLOT = 8

_LANES = 16
_UNROLL = 16
_ITERS = _CL // (_LANES * _UNROLL)


def _scale_buf(buf):
    def body(it, _):
        base = it * (_LANES * _UNROLL)
        for u in range(_UNROLL):
            o = base + u * _LANES
            buf[0, pl.ds(o, _LANES)] = buf[0, pl.ds(o, _LANES)] * _W
        return 0

    lax.fori_loop(0, _ITERS, body, 0)


def _sc_body(xf, didx, out2, idx_v, bufs, rsem, wsem):
    wid = lax.axis_index("s") * _NC + lax.axis_index("c")
    b = lax.div(wid, _GPB)
    g = lax.rem(wid, _GPB)
    row0 = (b * _IN_C + g * _CPW) * _NCHUNK

    pltpu.sync_copy(didx.at[wid], idx_v)

    def read_cp(s, slot):
        return pltpu.make_async_copy(
            xf.at[pl.ds(row0 + s, 1)], bufs.at[slot], rsem.at[slot]
        )

    def write_cp(s, slot, half):
        return pltpu.make_async_copy(
            bufs.at[slot],
            out2.at[idx_v.at[2 * s + half]],
            wsem.at[slot, half],
        )

    def process(s, slot):
        read_cp(s, slot).wait()
        _scale_buf(bufs.at[slot])
        write_cp(s, slot, 0).start()
        write_cp(s, slot, 1).start()

    for s in range(_DEPTH):
        read_cp(s, s % _NSLOT).start()
    for s in range(_DEPTH):
        read_cp(s + _DEPTH, (s + _DEPTH) % _NSLOT).start()
        process(s, s % _NSLOT)

    def steady(s, _):
        slot = lax.rem(s, _NSLOT)
        pslot = lax.rem(s + _DEPTH, _NSLOT)
        write_cp(s - _DEPTH, pslot, 0).wait()
        write_cp(s - _DEPTH, pslot, 1).wait()
        read_cp(s + _DEPTH, pslot).start()
        process(s, slot)
        return 0

    lax.fori_loop(_DEPTH, _STEPS - _DEPTH, steady, 0)

    for s in range(_STEPS - _DEPTH, _STEPS):
        process(s, s % _NSLOT)
    for s in range(_STEPS - _NSLOT, _STEPS):
        write_cp(s, s % _NSLOT, 0).wait()
        write_cp(s, s % _NSLOT, 1).wait()


def kernel(x, perm):
    B, C, H, W = x.shape
    xf = x.reshape(B * C * _NCHUNK, _CL)

    perm32 = perm.astype(jnp.int32)
    ar = jnp.arange(_IN_C, dtype=jnp.int32)
    z = jnp.zeros((_IN_C,), jnp.int32)
    dest0 = z.at[perm32[:_IN_C]].set(ar)
    dest1 = z.at[perm32[_IN_C:]].set(ar) + _IN_C

    wids = jnp.arange(_NW, dtype=jnp.int32)
    bs = wids // _GPB
    chs = ((wids % _GPB) * _CPW)[:, None] + ar[None, :_CPW]
    cr = jnp.arange(_NCHUNK, dtype=jnp.int32)
    rows0 = ((bs[:, None] * _OUT_C + dest0[chs]) * _NCHUNK)[:, :, None] + cr
    rows1 = ((bs[:, None] * _OUT_C + dest1[chs]) * _NCHUNK)[:, :, None] + cr
    didx = jnp.stack([rows0, rows1], axis=-1).reshape(_NW, 2 * _STEPS, 1)

    mesh = plsc.VectorSubcoreMesh(core_axis_name="c", subcore_axis_name="s")
    sc_call = pl.kernel(
        _sc_body,
        mesh=mesh,
        out_type=jax.ShapeDtypeStruct((B * _OUT_C * _NCHUNK, _CL), x.dtype),
        scratch_types=[
            pltpu.VMEM((2 * _STEPS, 1), jnp.int32),
            pltpu.VMEM((_NSLOT, 1, _CL), jnp.float32),
            pltpu.SemaphoreType.DMA((_NSLOT,)),
            pltpu.SemaphoreType.DMA((_NSLOT, 2)),
        ],
    )
    out = sc_call(xf, didx)
    return out.reshape(B, _OUT_C, H, W)


# speedup vs baseline: 1.8989x; 1.8989x over previous
import jax
import jax.numpy as jnp
from jax.experimental import pallas as pl
from jax.experimental.pallas import tpu as pltpu

_IN_C = 192
_OUT_C = 384
_W = 0.5
_NSLOT = 2
_CPB = 8
_STEPS = _IN_C // _CPB


def _body(dest_ref, x_ref, out_ref, scratch, sem):
    i = pl.program_id(0)
    slot = jax.lax.rem(i, _NSLOT)

    def _copies(st, s):
        cs = []
        for k in range(_CPB):
            ch = st * _CPB + k
            for half in range(2):
                d = dest_ref[half * _IN_C + ch]
                cs.append(
                    pltpu.make_async_copy(
                        scratch.at[s, :, pl.ds(k, 1)],
                        out_ref.at[:, pl.ds(d, 1)],
                        sem.at[s, 2 * k + half],
                    )
                )
        return cs

    @pl.when(i >= _NSLOT)
    def _():
        for c in _copies(i - _NSLOT, slot):
            c.wait()

    scratch[slot] = x_ref[...] * _W

    for c in _copies(i, slot):
        c.start()

    @pl.when(i == _STEPS - 1)
    def _():
        for back in range(_NSLOT - 1, -1, -1):
            for c in _copies(i - back, jax.lax.rem(i - back, _NSLOT)):
                c.wait()


def kernel(x, perm):
    B, C, H, W = x.shape
    HW = H * W
    S = HW // 128
    xr = x.reshape(B, C, S, 128)

    perm32 = perm.astype(jnp.int32)
    ar = jnp.arange(_IN_C, dtype=jnp.int32)
    z = jnp.zeros((_IN_C,), jnp.int32)
    dest0 = z.at[perm32[:_IN_C]].set(ar)
    dest1 = z.at[perm32[_IN_C:]].set(ar) + _IN_C
    dests = jnp.concatenate([dest0, dest1])

    out = pl.pallas_call(
        _body,
        grid_spec=pltpu.PrefetchScalarGridSpec(
            num_scalar_prefetch=1,
            grid=(_STEPS,),
            in_specs=[
                pl.BlockSpec(
                    (B, _CPB, S, 128), lambda i, dest_ref: (0, i, 0, 0)
                )
            ],
            out_specs=pl.BlockSpec(memory_space=pl.MemorySpace.ANY),
            scratch_shapes=[
                pltpu.VMEM((_NSLOT, B, _CPB, S, 128), jnp.float32),
                pltpu.SemaphoreType.DMA((_NSLOT, 2 * _CPB)),
            ],
        ),
        out_shape=jax.ShapeDtypeStruct((B, _OUT_C, S, 128), x.dtype),
    )(dests, xr)
    return out.reshape(B, _OUT_C, H, W)
